# baseline (speedup 1.0000x reference)
import functools

import jax
import jax.numpy as jnp
from jax import lax
from jax.experimental import pallas as pl
from jax.experimental.pallas import tpu as pltpu
from jax.experimental.pallas import tpu_sc as plsc

B = 4096
L = 200
S = 3
D = 64
N = B * L
NC = 2
NS = 16
NW = NC * NS
IB = 128
K = 2
CHUNK = K * IB
TOK_PER_W = N // NW
NCHUNK = TOK_PER_W // CHUNK
UNROLL = 4
LANES = 16


def _embed_sum(table_hbm, idx_hbm, out_hbm, raw_v, idx_v, out_v,
               isem0, isem1, isem2, isem3, gsem0, gsem1, asem0, asem1,
               osem0, osem1):
    wid = lax.axis_index("s") * NC + lax.axis_index("c")
    tok0 = wid * TOK_PER_W
    isems = (isem0, isem1, isem2, isem3)
    gsems = (gsem0, gsem1)
    asems = (asem0, asem1)
    osems = (osem0, osem1)
    iota3 = lax.iota(jnp.int32, LANES) * S

    def raw_copy(c, ph):
        return pltpu.make_async_copy(
            idx_hbm.at[pl.ds((tok0 + c * CHUNK) * S, S * CHUNK)],
            raw_v.at[ph], isems[ph])

    def restage(ph):
        src = raw_v.at[ph]
        gpb = IB // LANES

        def rbody(g, carry):
            base = g * LANES * S
            gi = g // gpb
            go = (g % gpb) * LANES
            for s in range(S):
                v = plsc.load_gather(src, [iota3 + (base + s)])
                idx_v[ph, s, gi, pl.ds(go, LANES)] = v
            return carry

        lax.fori_loop(0, CHUNK // LANES, rbody, 0)

    def gath0(c, ph, p):
        return [pltpu.make_async_copy(
                    table_hbm.at[idx_v.at[ph, 0, k]],
                    out_v.at[p, pl.ds(k * IB, IB)], gsems[p])
                for k in range(K)]

    def gath_add_start(ph, p):
        for s in (1, 2):
            for k in range(K):
                pltpu.async_copy(
                    table_hbm.at[idx_v.at[ph, s, k]],
                    out_v.at[p, pl.ds(k * IB, IB)], asems[p], add=True)

    def gath_add_wait(ph, p):
        for s in (1, 2):
            for k in range(K):
                pltpu.make_async_copy(
                    table_hbm.at[idx_v.at[ph, s, k]],
                    out_v.at[p, pl.ds(k * IB, IB)], asems[p]).wait()

    def out_copy(c, p):
        return pltpu.make_async_copy(
            out_v.at[p], out_hbm.at[pl.ds(tok0 + c * CHUNK, CHUNK)], osems[p])

    raw_copy(0, 0).start()
    raw_copy(1, 1).start()
    raw_copy(0, 0).wait()
    restage(0)
    for d in gath0(0, 0, 0):
        d.start()

    def step(c, p, ph, first, last, pf_idx=True):
        q = 1 - p
        phn = (ph + 1) % UNROLL
        if not last:
            if not first:
                out_copy(c - 1, q).wait()
            raw_copy(c + 1, phn).wait()
            restage(phn)
            for d in gath0(c + 1, phn, q):
                d.start()
        for d in gath0(c, ph, p):
            d.wait()
        gath_add_start(ph, p)
        if pf_idx:
            raw_copy(c + 2, (ph + 2) % UNROLL).start()
        gath_add_wait(ph, p)
        out_copy(c, p).start()

    def body(i, carry):
        for u in range(UNROLL):
            step(UNROLL * i + u, u % 2, u, False, False)
        return carry

    step(0, 0, 0, True, False)
    step(1, 1, 1, False, False)
    step(2, 0, 2, False, False)
    step(3, 1, 3, False, False)
    lax.fori_loop(1, NCHUNK // UNROLL - 1, body, 0)
    step(NCHUNK - 4, 0, 0, False, False)
    step(NCHUNK - 3, 1, 1, False, False)
    step(NCHUNK - 2, 0, 2, False, False, pf_idx=False)
    step(NCHUNK - 1, 1, 3, False, True, pf_idx=False)
    out_copy(NCHUNK - 2, 0).wait()
    out_copy(NCHUNK - 1, 1).wait()


@jax.jit
def _call(table, idx_flat):
    mesh = plsc.VectorSubcoreMesh(core_axis_name="c", subcore_axis_name="s")
    run = functools.partial(
        pl.kernel,
        out_type=jax.ShapeDtypeStruct((N, D), jnp.float32),
        mesh=mesh,
        compiler_params=pltpu.CompilerParams(
            use_tc_tiling_on_sc=False, needs_layout_passes=False),
        scratch_types=[
            pltpu.VMEM((UNROLL, S * CHUNK), jnp.int32),
            pltpu.VMEM((UNROLL, S, K, IB), jnp.int32),
            pltpu.VMEM((2, CHUNK, D), jnp.float32),
        ] + [pltpu.SemaphoreType.DMA] * 10,
    )(_embed_sum)
    return run(table, idx_flat)


def kernel(indices, table):
    idx_flat = indices.astype(jnp.int32).reshape(N * S)
    out = _call(table, idx_flat)
    return out.reshape(B, L, D)


# speedup vs baseline: 2.6693x; 2.6693x over previous
import functools

import jax
import jax.numpy as jnp
from jax import lax
from jax.experimental import pallas as pl
from jax.experimental.pallas import tpu as pltpu
from jax.experimental.pallas import tpu_sc as plsc

B = 4096
L = 200
S = 3
D = 64
N = B * L
NC = 2
NS = 16
NW = NC * NS
IB = 128
K = 2
CHUNK = K * IB
TOK_PER_W = N // NW
NCHUNK = TOK_PER_W // CHUNK
UNROLL = 4
LANES = 16


def _embed_sum(table_hbm, idx_hbm, out_hbm, raw_v, idx_v, out_v,
               isem0, isem1, isem2, isem3, gsem0, gsem1, asem0, asem1,
               osem0, osem1):
    wid = lax.axis_index("s") * NC + lax.axis_index("c")
    tok0 = wid * TOK_PER_W
    isems = (isem0, isem1, isem2, isem3)
    gsems = (gsem0, gsem1)
    asems = (asem0, asem1)
    osems = (osem0, osem1)
    iota3 = lax.iota(jnp.int32, LANES) * S

    def raw_copy(c, ph):
        blk0 = wid * (TOK_PER_W // IB)
        return pltpu.make_async_copy(
            idx_hbm.at[:, pl.ds(blk0 + c * K, K), :], idx_v.at[ph], isems[ph])

    def restage(ph):
        src = raw_v.at[ph]
        gpb = IB // LANES

        def rbody(g, carry):
            base = g * LANES * S
            gi = g // gpb
            go = (g % gpb) * LANES
            for s in range(S):
                v = plsc.load_gather(src, [iota3 + (base + s)])
                idx_v[ph, s, gi, pl.ds(go, LANES)] = v
            return carry

        if False:
            lax.fori_loop(0, CHUNK // LANES, rbody, 0)

    def gath0(c, ph, p):
        return [pltpu.make_async_copy(
                    table_hbm.at[idx_v.at[ph, 0, k]],
                    out_v.at[p, pl.ds(k * IB, IB)], gsems[p])
                for k in range(K)]

    def gath_add_start(ph, p):
        for s in (1, 2):
            for k in range(K):
                pltpu.async_copy(
                    table_hbm.at[idx_v.at[ph, s, k]],
                    out_v.at[p, pl.ds(k * IB, IB)], asems[p], add=True)

    def gath_add_wait(ph, p):
        for s in (1, 2):
            for k in range(K):
                pltpu.make_async_copy(
                    table_hbm.at[idx_v.at[ph, s, k]],
                    out_v.at[p, pl.ds(k * IB, IB)], asems[p]).wait()

    def out_copy(c, p):
        return pltpu.make_async_copy(
            out_v.at[p], out_hbm.at[pl.ds(tok0 + c * CHUNK, CHUNK)], osems[p])

    raw_copy(0, 0).start()
    raw_copy(1, 1).start()
    raw_copy(0, 0).wait()
    restage(0)
    for d in gath0(0, 0, 0):
        d.start()

    def step(c, p, ph, first, last, pf_idx=True):
        q = 1 - p
        phn = (ph + 1) % UNROLL
        if not last:
            if not first:
                out_copy(c - 1, q).wait()
            raw_copy(c + 1, phn).wait()
            restage(phn)
            for d in gath0(c + 1, phn, q):
                d.start()
        for d in gath0(c, ph, p):
            d.wait()
        gath_add_start(ph, p)
        if pf_idx:
            raw_copy(c + 2, (ph + 2) % UNROLL).start()
        gath_add_wait(ph, p)
        out_copy(c, p).start()

    def body(i, carry):
        for u in range(UNROLL):
            step(UNROLL * i + u, u % 2, u, False, False)
        return carry

    step(0, 0, 0, True, False)
    step(1, 1, 1, False, False)
    step(2, 0, 2, False, False)
    step(3, 1, 3, False, False)
    lax.fori_loop(1, NCHUNK // UNROLL - 1, body, 0)
    step(NCHUNK - 4, 0, 0, False, False)
    step(NCHUNK - 3, 1, 1, False, False)
    step(NCHUNK - 2, 0, 2, False, False, pf_idx=False)
    step(NCHUNK - 1, 1, 3, False, True, pf_idx=False)
    out_copy(NCHUNK - 2, 0).wait()
    out_copy(NCHUNK - 1, 1).wait()


@jax.jit
def _call(table, idx_flat):
    mesh = plsc.VectorSubcoreMesh(core_axis_name="c", subcore_axis_name="s")
    run = functools.partial(
        pl.kernel,
        out_type=jax.ShapeDtypeStruct((N, D), jnp.float32),
        mesh=mesh,
        compiler_params=pltpu.CompilerParams(
            use_tc_tiling_on_sc=False, needs_layout_passes=False),
        scratch_types=[
            pltpu.VMEM((UNROLL, S * CHUNK), jnp.int32),
            pltpu.VMEM((UNROLL, S, K, IB), jnp.int32),
            pltpu.VMEM((2, CHUNK, D), jnp.float32),
        ] + [pltpu.SemaphoreType.DMA] * 10,
    )(_embed_sum)
    return run(table, idx_flat)


def kernel(indices, table):
    idx_r = indices.astype(jnp.int32).reshape(N, S).T.reshape(S, N // IB, IB)
    out = _call(table, idx_r)
    return out.reshape(B, L, D)
